# Initial kernel scaffold; baseline (speedup 1.0000x reference)
import jax
import jax.numpy as jnp
from jax.experimental import pallas as pl


def kernel(x, edge_index, edge_attr, batch, We, W1, b1, W2, b2, Wv, bv):
    raise NotImplementedError("write your pallas kernel here")


# speedup vs baseline: 2.6142x; 2.6142x over previous
import functools

import jax
import jax.numpy as jnp
from jax import lax
from jax.experimental import pallas as pl
from jax.experimental.pallas import tpu as pltpu
from jax.experimental.pallas import tpu_sc as plsc

N_NODES = 10000
N_EDGES = 320000
D = 128
HID = 256
G = 512
LAYERS = 5

NC = 2
NS = 16
NW = NC * NS
EPW = N_EDGES // NW
CH = 80
NCHUNK = EPW // CH
RPT = 624
ZR = 208


@functools.lru_cache(maxsize=1)
def _sc_msgpass():
    mesh = plsc.VectorSubcoreMesh(core_axis_name="c", subcore_axis_name="s",
                                  num_cores=NC, num_subcores=NS)

    @functools.partial(
        pl.kernel,
        out_type=jax.ShapeDtypeStruct((NC, N_NODES, D), jnp.float32),
        mesh=mesh,
        scratch_types=[
            pltpu.VMEM((CH,), jnp.int32),
            pltpu.VMEM((CH,), jnp.int32),
            pltpu.VMEM((CH, D), jnp.float32),
            pltpu.VMEM((CH, D), jnp.float32),
            pltpu.VMEM((ZR, D), jnp.float32),
            pltpu.VMEM_SHARED((N_NODES, D), jnp.float32),
            pltpu.SemaphoreType.DMA,
        ],
    )
    def k(hin_hbm, e_hbm, src_hbm, dst_hbm, out_hbm,
          src_v, dst_v, hbuf, ebuf, zbuf, agg_sh, sem):
        cid = lax.axis_index("c")
        sid = lax.axis_index("s")
        wid = cid * NS + sid

        zero16 = jnp.zeros((16,), jnp.float32)

        def zrow(r, _):
            for c in range(D // 16):
                zbuf[r, pl.ds(c * 16, 16)] = zero16
            return 0

        lax.fori_loop(0, ZR, zrow, 0)
        for j in range(RPT // ZR):
            pltpu.sync_copy(zbuf, agg_sh.at[pl.ds(sid * RPT + j * ZR, ZR)])

        @pl.when(sid == NS - 1)
        def _():
            pltpu.sync_copy(zbuf.at[pl.ds(0, 16)],
                            agg_sh.at[pl.ds(NS * RPT, 16)])

        plsc.subcore_barrier()

        def chunk(c, _):
            base = wid * EPW + c * CH
            pltpu.sync_copy(src_hbm.at[pl.ds(base, CH)], src_v)
            pltpu.sync_copy(dst_hbm.at[pl.ds(base, CH)], dst_v)
            pltpu.async_copy(hin_hbm.at[src_v], hbuf, sem).wait()
            pltpu.sync_copy(e_hbm.at[pl.ds(base, CH)], ebuf)

            def row(r, _):
                for cc in range(D // 16):
                    s = pl.ds(cc * 16, 16)
                    hbuf[r, s] = jnp.maximum(hbuf[r, s] + ebuf[r, s], 0.0)
                return 0

            lax.fori_loop(0, CH, row, 0)
            pltpu.sync_copy(hbuf, agg_sh.at[dst_v], add=True)
            return 0

        lax.fori_loop(0, NCHUNK, chunk, 0)
        plsc.subcore_barrier()

        pltpu.sync_copy(agg_sh.at[pl.ds(sid * RPT, RPT)],
                        out_hbm.at[cid, pl.ds(sid * RPT, RPT)])

        @pl.when(sid == NS - 1)
        def _():
            pltpu.sync_copy(agg_sh.at[pl.ds(NS * RPT, 16)],
                            out_hbm.at[cid, pl.ds(NS * RPT, 16)])

    return k


RB = 2000
NRB = N_NODES // RB
EB = 8000
NEB = N_EDGES // EB


def _p_body(batch_ref, p_ref):
    ids = lax.broadcasted_iota(jnp.int32, (RB, G), 1)
    p_ref[...] = (batch_ref[...] == ids).astype(jnp.float32)


def _build_p(batch):
    return pl.pallas_call(
        _p_body,
        grid=(NRB,),
        in_specs=[pl.BlockSpec((RB, 1), lambda i: (i, 0))],
        out_specs=pl.BlockSpec((RB, G), lambda i: (i, 0)),
        out_shape=jax.ShapeDtypeStruct((N_NODES, G), jnp.float32),
    )(batch.reshape(N_NODES, 1))


def _e_body(ea_ref, we_ref, e_ref):
    e_ref[...] = jnp.maximum(
        jnp.dot(ea_ref[...], we_ref[...], preferred_element_type=jnp.float32),
        0.0)


def _bond_encode(edge_attr, we):
    return pl.pallas_call(
        _e_body,
        grid=(NEB,),
        in_specs=[pl.BlockSpec((EB, 16), lambda i: (i, 0)),
                  pl.BlockSpec((16, D), lambda i: (0, 0))],
        out_specs=pl.BlockSpec((EB, D), lambda i: (i, 0)),
        out_shape=jax.ShapeDtypeStruct((N_EDGES, D), jnp.float32),
    )(edge_attr, we)


def _u1_body(agg2_ref, hin_ref, p_ref, w1_ref, b1_ref, w2_ref, b2_ref,
             hnew_ref, pooled_ref):
    agg = agg2_ref[0] + agg2_ref[1]
    t = jnp.maximum(
        jnp.dot(agg, w1_ref[...], preferred_element_type=jnp.float32)
        + b1_ref[...], 0.0)
    out = jnp.dot(t, w2_ref[...], preferred_element_type=jnp.float32) \
        + b2_ref[...]
    hnew = hin_ref[...] + out
    hnew_ref[...] = hnew
    part = lax.dot_general(p_ref[...], hnew, (((0,), (0,)), ((), ())),
                           preferred_element_type=jnp.float32)

    @pl.when(pl.program_id(0) == 0)
    def _():
        pooled_ref[...] = jnp.zeros_like(pooled_ref)

    pooled_ref[...] += part


def _layer_update(agg2, hin, p, w1, b1, w2, b2):
    return pl.pallas_call(
        _u1_body,
        grid=(NRB,),
        in_specs=[
            pl.BlockSpec((NC, RB, D), lambda i: (0, i, 0)),
            pl.BlockSpec((RB, D), lambda i: (i, 0)),
            pl.BlockSpec((RB, G), lambda i: (i, 0)),
            pl.BlockSpec((D, HID), lambda i: (0, 0)),
            pl.BlockSpec((1, HID), lambda i: (0, 0)),
            pl.BlockSpec((HID, D), lambda i: (0, 0)),
            pl.BlockSpec((1, D), lambda i: (0, 0)),
        ],
        out_specs=[
            pl.BlockSpec((RB, D), lambda i: (i, 0)),
            pl.BlockSpec((G, D), lambda i: (0, 0)),
        ],
        out_shape=[
            jax.ShapeDtypeStruct((N_NODES, D), jnp.float32),
            jax.ShapeDtypeStruct((G, D), jnp.float32),
        ],
    )(agg2, hin, p, w1, b1.reshape(1, HID), w2, b2.reshape(1, D))


def _virt_body(pooled_ref, virt_ref, wv_ref, bv_ref, virtnew_ref):
    virtnew_ref[...] = virt_ref[...] + jnp.maximum(
        jnp.dot(pooled_ref[...], wv_ref[...],
                preferred_element_type=jnp.float32) + bv_ref[...], 0.0)


def _virt_update(pooled, virt, wv, bv):
    return pl.pallas_call(
        _virt_body,
        out_shape=jax.ShapeDtypeStruct((G, D), jnp.float32),
    )(pooled, virt, wv, bv.reshape(1, D))


def _bcast_body(hnew_ref, p_ref, virt_ref, hin_ref):
    hin_ref[...] = hnew_ref[...] + jnp.dot(
        p_ref[...], virt_ref[...], preferred_element_type=jnp.float32)


def _virt_broadcast(hnew, p, virtnew):
    return pl.pallas_call(
        _bcast_body,
        grid=(NRB,),
        in_specs=[pl.BlockSpec((RB, D), lambda i: (i, 0)),
                  pl.BlockSpec((RB, G), lambda i: (i, 0)),
                  pl.BlockSpec((G, D), lambda i: (0, 0))],
        out_specs=pl.BlockSpec((RB, D), lambda i: (i, 0)),
        out_shape=jax.ShapeDtypeStruct((N_NODES, D), jnp.float32),
    )(hnew, p, virtnew)


def _norm_body(pooled_ref, hg_ref):
    p = pooled_ref[...]
    m = jnp.mean(p, axis=1, keepdims=True)
    v = jnp.mean((p - m) ** 2, axis=1, keepdims=True)
    hg_ref[...] = (p - m) * lax.rsqrt(v + 1e-5)


def _group_norm(pooled):
    return pl.pallas_call(
        _norm_body,
        out_shape=jax.ShapeDtypeStruct((G, D), jnp.float32),
    )(pooled)


def kernel(x, edge_index, edge_attr, batch, We, W1, b1, W2, b2, Wv, bv):
    src = edge_index[0]
    dst = edge_index[1]
    p = _build_p(batch)
    hin = x
    virt = jnp.zeros((G, D), jnp.float32)
    pooled = None
    for l in range(LAYERS):
        e = _bond_encode(edge_attr, We[l])
        agg2 = _sc_msgpass()(hin, e, src, dst)
        hnew, pooled = _layer_update(agg2, hin, p, W1[l], b1[l], W2[l], b2[l])
        if l < LAYERS - 1:
            virt = _virt_update(pooled, virt, Wv[l], bv[l])
            hin = _virt_broadcast(hnew, p, virt)
    return _group_norm(pooled)


# speedup vs baseline: 4.9105x; 1.8784x over previous
import functools

import jax
import jax.numpy as jnp
from jax import lax
from jax.experimental import pallas as pl
from jax.experimental.pallas import tpu as pltpu
from jax.experimental.pallas import tpu_sc as plsc

N_NODES = 10000
N_EDGES = 320000
D = 128
HID = 256
G = 512
LAYERS = 5

NC = 2
NS = 16
NW = NC * NS
EPW = N_EDGES // NW
CH = 40
NCHUNK = EPW // CH
SUP = 50
NSUP = NCHUNK // SUP
RPT = 624
ZR = 208


@functools.lru_cache(maxsize=1)
def _sc_msgpass():
    mesh = plsc.VectorSubcoreMesh(core_axis_name="c", subcore_axis_name="s",
                                  num_cores=NC, num_subcores=NS)

    @functools.partial(
        pl.kernel,
        out_type=jax.ShapeDtypeStruct((NC, N_NODES, D), jnp.float32),
        mesh=mesh,
        scratch_types=[
            pltpu.VMEM((SUP, CH), jnp.int32),
            pltpu.VMEM((SUP, CH), jnp.int32),
            pltpu.VMEM((2, CH, D), jnp.float32),
            pltpu.VMEM((2, CH, D), jnp.float32),
            pltpu.VMEM((2, CH, D), jnp.float32),
            pltpu.VMEM_SHARED((N_NODES, D), jnp.float32),
            pltpu.SemaphoreType.DMA,
            pltpu.SemaphoreType.DMA,
            pltpu.SemaphoreType.DMA,
            pltpu.SemaphoreType.DMA,
            pltpu.SemaphoreType.DMA,
            pltpu.SemaphoreType.DMA,
        ],
    )
    def k(hin_hbm, e_hbm, srcs_hbm, dsts_hbm, out_hbm,
          src_v, dst_v, hbuf, ebuf, mbuf, agg_sh,
          g0, g1, e0, e1, s0, s1):
        gsem = (g0, g1)
        esem = (e0, e1)
        ssem = (s0, s1)
        cid = lax.axis_index("c")
        sid = lax.axis_index("s")
        wid = cid * NS + sid

        zero16 = jnp.zeros((16,), jnp.float32)

        def zrow(r, _):
            for c in range(D // 16):
                mbuf[0, r, pl.ds(c * 16, 16)] = zero16
            return 0

        lax.fori_loop(0, CH, zrow, 0)
        for j in range(RPT // 40):
            pltpu.sync_copy(mbuf.at[0],
                            agg_sh.at[pl.ds(sid * RPT + j * 40, 40)])
        pltpu.sync_copy(mbuf.at[0, pl.ds(0, 24)],
                        agg_sh.at[pl.ds(sid * RPT + 600, 24)])

        @pl.when(sid == NS - 1)
        def _():
            pltpu.sync_copy(mbuf.at[0, pl.ds(0, 16)],
                            agg_sh.at[pl.ds(NS * RPT, 16)])

        plsc.subcore_barrier()

        def issue(sup, c, b):
            base = wid * EPW + (sup * SUP + c) * CH
            pltpu.async_copy(hin_hbm.at[src_v.at[c]], hbuf.at[b], gsem[b])
            pltpu.async_copy(e_hbm.at[pl.ds(base, CH)], ebuf.at[b], esem[b])

        def process(sup, c, b, first_pair):
            base = wid * EPW + (sup * SUP + c) * CH
            pltpu.make_async_copy(hin_hbm.at[src_v.at[c]], hbuf.at[b],
                                  gsem[b]).wait()
            pltpu.make_async_copy(e_hbm.at[pl.ds(base, CH)], ebuf.at[b],
                                  esem[b]).wait()

            if not first_pair:
                pltpu.make_async_copy(mbuf.at[b], agg_sh.at[dst_v.at[c]],
                                      ssem[b]).wait()

            @plsc.parallel_loop(0, CH, 1, unroll=2)
            def _(r):
                for cc in range(D // 16):
                    s = pl.ds(cc * 16, 16)
                    mbuf[b, r, s] = jnp.maximum(
                        hbuf[b, r, s] + ebuf[b, r, s], 0.0)

            pltpu.async_copy(mbuf.at[b], agg_sh.at[dst_v.at[c]], ssem[b],
                             add=True)

        for sup in range(NSUP):
            if sup > 0:
                pltpu.make_async_copy(mbuf.at[0], agg_sh.at[dst_v.at[0]],
                                      ssem[0]).wait()
                pltpu.make_async_copy(mbuf.at[1], agg_sh.at[dst_v.at[0]],
                                      ssem[1]).wait()
            pltpu.sync_copy(srcs_hbm.at[wid, sup], src_v)
            pltpu.sync_copy(dsts_hbm.at[wid, sup], dst_v)
            issue(sup, 0, 0)
            issue(sup, 1, 1)
            process(sup, 0, 0, True)
            issue(sup, 2, 0)
            process(sup, 1, 1, True)
            issue(sup, 3, 1)

            @pl.loop(2, SUP - 2, step=2)
            def _(c0):
                process(sup, c0, 0, False)
                issue(sup, c0 + 2, 0)
                process(sup, c0 + 1, 1, False)
                issue(sup, c0 + 3, 1)

            process(sup, SUP - 2, 0, False)
            process(sup, SUP - 1, 1, False)

        pltpu.make_async_copy(mbuf.at[0], agg_sh.at[dst_v.at[0]],
                              ssem[0]).wait()
        pltpu.make_async_copy(mbuf.at[1], agg_sh.at[dst_v.at[0]],
                              ssem[1]).wait()
        plsc.subcore_barrier()

        pltpu.sync_copy(agg_sh.at[pl.ds(sid * RPT, RPT)],
                        out_hbm.at[cid, pl.ds(sid * RPT, RPT)])

        @pl.when(sid == NS - 1)
        def _():
            pltpu.sync_copy(agg_sh.at[pl.ds(NS * RPT, 16)],
                            out_hbm.at[cid, pl.ds(NS * RPT, 16)])

    return k


RB = 2000
NRB = N_NODES // RB
EB = 8000
NEB = N_EDGES // EB


def _p_body(batch_ref, p_ref):
    ids = lax.broadcasted_iota(jnp.int32, (RB, G), 1)
    p_ref[...] = (batch_ref[...] == ids).astype(jnp.float32)


def _build_p(batch):
    return pl.pallas_call(
        _p_body,
        grid=(NRB,),
        in_specs=[pl.BlockSpec((RB, 1), lambda i: (i, 0))],
        out_specs=pl.BlockSpec((RB, G), lambda i: (i, 0)),
        out_shape=jax.ShapeDtypeStruct((N_NODES, G), jnp.float32),
    )(batch.reshape(N_NODES, 1))


def _e_body(ea_ref, we_ref, e_ref):
    e_ref[...] = jnp.maximum(
        jnp.dot(ea_ref[...], we_ref[...], preferred_element_type=jnp.float32),
        0.0)


def _bond_encode(edge_attr, we):
    return pl.pallas_call(
        _e_body,
        grid=(NEB,),
        in_specs=[pl.BlockSpec((EB, 16), lambda i: (i, 0)),
                  pl.BlockSpec((16, D), lambda i: (0, 0))],
        out_specs=pl.BlockSpec((EB, D), lambda i: (i, 0)),
        out_shape=jax.ShapeDtypeStruct((N_EDGES, D), jnp.float32),
    )(edge_attr, we)


def _u1_body(agg2_ref, hin_ref, p_ref, w1_ref, b1_ref, w2_ref, b2_ref,
             hnew_ref, pooled_ref):
    agg = agg2_ref[0] + agg2_ref[1]
    t = jnp.maximum(
        jnp.dot(agg, w1_ref[...], preferred_element_type=jnp.float32)
        + b1_ref[...], 0.0)
    out = jnp.dot(t, w2_ref[...], preferred_element_type=jnp.float32) \
        + b2_ref[...]
    hnew = hin_ref[...] + out
    hnew_ref[...] = hnew
    part = lax.dot_general(p_ref[...], hnew, (((0,), (0,)), ((), ())),
                           preferred_element_type=jnp.float32)

    @pl.when(pl.program_id(0) == 0)
    def _():
        pooled_ref[...] = jnp.zeros_like(pooled_ref)

    pooled_ref[...] += part


def _layer_update(agg2, hin, p, w1, b1, w2, b2):
    return pl.pallas_call(
        _u1_body,
        grid=(NRB,),
        in_specs=[
            pl.BlockSpec((NC, RB, D), lambda i: (0, i, 0)),
            pl.BlockSpec((RB, D), lambda i: (i, 0)),
            pl.BlockSpec((RB, G), lambda i: (i, 0)),
            pl.BlockSpec((D, HID), lambda i: (0, 0)),
            pl.BlockSpec((1, HID), lambda i: (0, 0)),
            pl.BlockSpec((HID, D), lambda i: (0, 0)),
            pl.BlockSpec((1, D), lambda i: (0, 0)),
        ],
        out_specs=[
            pl.BlockSpec((RB, D), lambda i: (i, 0)),
            pl.BlockSpec((G, D), lambda i: (0, 0)),
        ],
        out_shape=[
            jax.ShapeDtypeStruct((N_NODES, D), jnp.float32),
            jax.ShapeDtypeStruct((G, D), jnp.float32),
        ],
    )(agg2, hin, p, w1, b1.reshape(1, HID), w2, b2.reshape(1, D))


def _virt_body(pooled_ref, virt_ref, wv_ref, bv_ref, virtnew_ref):
    virtnew_ref[...] = virt_ref[...] + jnp.maximum(
        jnp.dot(pooled_ref[...], wv_ref[...],
                preferred_element_type=jnp.float32) + bv_ref[...], 0.0)


def _virt_update(pooled, virt, wv, bv):
    return pl.pallas_call(
        _virt_body,
        out_shape=jax.ShapeDtypeStruct((G, D), jnp.float32),
    )(pooled, virt, wv, bv.reshape(1, D))


def _bcast_body(hnew_ref, p_ref, virt_ref, hin_ref):
    hin_ref[...] = hnew_ref[...] + jnp.dot(
        p_ref[...], virt_ref[...], preferred_element_type=jnp.float32)


def _virt_broadcast(hnew, p, virtnew):
    return pl.pallas_call(
        _bcast_body,
        grid=(NRB,),
        in_specs=[pl.BlockSpec((RB, D), lambda i: (i, 0)),
                  pl.BlockSpec((RB, G), lambda i: (i, 0)),
                  pl.BlockSpec((G, D), lambda i: (0, 0))],
        out_specs=pl.BlockSpec((RB, D), lambda i: (i, 0)),
        out_shape=jax.ShapeDtypeStruct((N_NODES, D), jnp.float32),
    )(hnew, p, virtnew)


def _norm_body(pooled_ref, hg_ref):
    p = pooled_ref[...]
    m = jnp.mean(p, axis=1, keepdims=True)
    v = jnp.mean((p - m) ** 2, axis=1, keepdims=True)
    hg_ref[...] = (p - m) * lax.rsqrt(v + 1e-5)


def _group_norm(pooled):
    return pl.pallas_call(
        _norm_body,
        out_shape=jax.ShapeDtypeStruct((G, D), jnp.float32),
    )(pooled)


def kernel(x, edge_index, edge_attr, batch, We, W1, b1, W2, b2, Wv, bv):
    src = edge_index[0].reshape(NW, NSUP, SUP, CH)
    dst = edge_index[1].reshape(NW, NSUP, SUP, CH)
    p = _build_p(batch)
    hin = x
    virt = jnp.zeros((G, D), jnp.float32)
    pooled = None
    for l in range(LAYERS):
        e = _bond_encode(edge_attr, We[l])
        agg2 = _sc_msgpass()(hin, e, src, dst)
        hnew, pooled = _layer_update(agg2, hin, p, W1[l], b1[l], W2[l], b2[l])
        if l < LAYERS - 1:
            virt = _virt_update(pooled, virt, Wv[l], bv[l])
            hin = _virt_broadcast(hnew, p, virt)
    return _group_norm(pooled)
